# baseline (speedup 1.0000x reference)
import jax
import jax.numpy as jnp
from jax import lax
from jax.experimental import pallas as pl

K = 16
L = 2048
N = 8
C = 64
TQ = 1024


def _oem_pairs(lo, n, r):
    m = r * 2
    if m < n:
        yield from _oem_pairs(lo, n, m)
        yield from _oem_pairs(lo + r, n, m)
        for i in range(lo + r, lo + n - r, m):
            yield (i, i + r)
    else:
        yield (lo, lo + r)


def _oems_pairs(lo, n):
    if n > 1:
        m = n // 2
        yield from _oems_pairs(lo, m)
        yield from _oems_pairs(lo + m, m)
        yield from _oem_pairs(lo, n, 1)


_CE_PAIRS = tuple(_oems_pairs(0, L // 128))


def _knn_tile(y_ref, xt_ref, out_ref):
    y = y_ref[0]
    xt = xt_ref[0]
    s = lax.dot_general(y, xt, (((1,), (0,)), ((), ())),
                        preferred_element_type=jnp.float32)
    ynorm = jnp.sum(y * y, axis=1, keepdims=True)
    xnorm = jnp.sum(xt * xt, axis=0, keepdims=True)
    d = ynorm - 2.0 * s + xnorm

    lane = lax.broadcasted_iota(jnp.int32, (TQ, 128), 1).astype(jnp.float32)
    S = [d[:, p * 128:(p + 1) * 128] for p in range(L // 128)]
    I = [lane + jnp.float32(p * 128) for p in range(L // 128)]
    for a, b in _CE_PAIRS:
        va, vb, ia, ib = S[a], S[b], I[a], I[b]
        swap = (vb < va) | ((vb == va) & (ib < ia))
        S[a] = jnp.where(swap, vb, va)
        S[b] = jnp.where(swap, va, vb)
        I[a] = jnp.where(swap, ib, ia)
        I[b] = jnp.where(swap, ia, ib)
    big = jnp.float32(L)
    cols = []
    for k in range(K):
        m = jnp.min(S[0], axis=1, keepdims=True)
        idx = jnp.min(jnp.where(S[0] == m, I[0], big), axis=1, keepdims=True)
        cols.append(idx)
        if k + 1 < K:
            eqlane = I[0] == idx
            for j in range(K - k - 1):
                S[j] = jnp.where(eqlane, S[j + 1], S[j])
                I[j] = jnp.where(eqlane, I[j + 1], I[j])
    out_ref[0] = jnp.concatenate(cols, axis=1).astype(jnp.int32)


def kernel(coords1, coords2):
    xt = jnp.transpose(coords1, (1, 2, 0))
    y = jnp.swapaxes(coords2, 0, 1)

    grid = (N, L // TQ)
    idx = pl.pallas_call(
        _knn_tile,
        grid=grid,
        in_specs=[
            pl.BlockSpec((1, TQ, C), lambda n, q: (n, q, 0)),
            pl.BlockSpec((1, C, L), lambda n, q: (n, 0, 0)),
        ],
        out_specs=pl.BlockSpec((1, TQ, K), lambda n, q: (n, q, 0)),
        out_shape=jax.ShapeDtypeStruct((N, L, K), jnp.int32),
    )(y, xt)

    clusters = jnp.transpose(idx, (2, 1, 0))
    indices0 = clusters.reshape(-1).astype(jnp.int64)
    batch_grid = jnp.broadcast_to(jnp.arange(N), (K, L, N))
    indices1 = batch_grid.reshape(-1).astype(jnp.int64)
    return (indices0, indices1)


# speedup vs baseline: 1.0272x; 1.0272x over previous
import jax
import jax.numpy as jnp
from jax import lax
from jax.experimental import pallas as pl

K = 16
L = 2048
N = 8
C = 64
TQ = 512


def _oem_pairs(lo, n, r):
    m = r * 2
    if m < n:
        yield from _oem_pairs(lo, n, m)
        yield from _oem_pairs(lo + r, n, m)
        for i in range(lo + r, lo + n - r, m):
            yield (i, i + r)
    else:
        yield (lo, lo + r)


def _oems_pairs(lo, n):
    if n > 1:
        m = n // 2
        yield from _oems_pairs(lo, m)
        yield from _oems_pairs(lo + m, m)
        yield from _oem_pairs(lo, n, 1)


_CE_PAIRS = tuple(_oems_pairs(0, L // 128))


def _knn_tile(y_ref, xt_ref, out_ref):
    y = y_ref[0]
    xt = xt_ref[0]
    s = lax.dot_general(y, xt, (((1,), (0,)), ((), ())),
                        preferred_element_type=jnp.float32)
    ynorm = jnp.sum(y * y, axis=1, keepdims=True)
    xnorm = jnp.sum(xt * xt, axis=0, keepdims=True)
    d = ynorm - 2.0 * s + xnorm

    lane = lax.broadcasted_iota(jnp.int32, (TQ, 128), 1).astype(jnp.float32)
    S = [d[:, p * 128:(p + 1) * 128] for p in range(L // 128)]
    I = [lane + jnp.float32(p * 128) for p in range(L // 128)]
    for a, b in _CE_PAIRS:
        va, vb, ia, ib = S[a], S[b], I[a], I[b]
        swap = (vb < va) | ((vb == va) & (ib < ia))
        S[a] = jnp.where(swap, vb, va)
        S[b] = jnp.where(swap, va, vb)
        I[a] = jnp.where(swap, ib, ia)
        I[b] = jnp.where(swap, ia, ib)
    big = jnp.float32(L)
    cols = []
    for k in range(K):
        m = jnp.min(S[0], axis=1, keepdims=True)
        idx = jnp.min(jnp.where(S[0] == m, I[0], big), axis=1, keepdims=True)
        cols.append(idx)
        if k + 1 < K:
            eqlane = I[0] == idx
            for j in range(K - k - 1):
                S[j] = jnp.where(eqlane, S[j + 1], S[j])
                I[j] = jnp.where(eqlane, I[j + 1], I[j])
    out_ref[0] = jnp.concatenate(cols, axis=1).astype(jnp.int32)


def kernel(coords1, coords2):
    xt = jnp.transpose(coords1, (1, 2, 0))
    y = jnp.swapaxes(coords2, 0, 1)

    grid = (N, L // TQ)
    idx = pl.pallas_call(
        _knn_tile,
        grid=grid,
        in_specs=[
            pl.BlockSpec((1, TQ, C), lambda n, q: (n, q, 0)),
            pl.BlockSpec((1, C, L), lambda n, q: (n, 0, 0)),
        ],
        out_specs=pl.BlockSpec((1, TQ, K), lambda n, q: (n, q, 0)),
        out_shape=jax.ShapeDtypeStruct((N, L, K), jnp.int32),
    )(y, xt)

    clusters = jnp.transpose(idx, (2, 1, 0))
    indices0 = clusters.reshape(-1).astype(jnp.int64)
    batch_grid = jnp.broadcast_to(jnp.arange(N), (K, L, N))
    indices1 = batch_grid.reshape(-1).astype(jnp.int64)
    return (indices0, indices1)
